# baseline (speedup 1.0000x reference)
import jax
import jax.numpy as jnp
from jax.experimental import pallas as pl
from jax.experimental.pallas import tpu as pltpu

D = 256
TM = 512

_DN = (((1,), (1,)), ((), ()))


def _combine_weights(wcl_ref, bcl_ref, wro_ref, bro_ref, wc_ref, bc_ref):
    wc = jax.lax.dot_general(
        wro_ref[...], wcl_ref[...], (((1,), (0,)), ((), ())),
        preferred_element_type=jnp.float32)
    wc_ref[...] = wc.astype(jnp.bfloat16)
    bc = jax.lax.dot_general(bcl_ref[...], wro_ref[...], _DN,
                             preferred_element_type=jnp.float32)
    bc_ref[...] = bc + bro_ref[...]


def _fused_rows(x_ref, win_ref, bin_ref, wro_ref, wc_ref, bc_ref, o_ref):
    x = x_ref[...].astype(jnp.bfloat16)
    t = jnp.tanh(jax.lax.dot_general(x, win_ref[...], _DN,
                                     preferred_element_type=jnp.float32)
                 + bin_ref[...])
    o = jax.lax.dot_general(t.astype(jnp.bfloat16), wro_ref[...], _DN,
                            preferred_element_type=jnp.float32)
    o += jax.lax.dot_general(x, wc_ref[...], _DN,
                             preferred_element_type=jnp.float32)
    o_ref[...] = o + bc_ref[...]


def kernel(x_patient, x_se, edge_index, W_in, b_in, W_se, b_se, W_cl, b_cl,
           W_ro, b_ro, Wl_0_pse, bl_0_pse, Wr_0_pse, Wl_0_rev, bl_0_rev,
           Wr_0_rev, Wl_1_pse, bl_1_pse, Wr_1_pse, Wl_1_rev, bl_1_rev,
           Wr_1_rev):
    wro_bf16 = W_ro.astype(jnp.bfloat16)
    w_comb, b_comb = pl.pallas_call(
        _combine_weights,
        out_shape=(jax.ShapeDtypeStruct((D, D), jnp.bfloat16),
                   jax.ShapeDtypeStruct((1, D), jnp.float32)),
    )(W_cl.astype(jnp.bfloat16), b_cl.reshape(1, D).astype(jnp.bfloat16),
      wro_bf16, b_ro.reshape(1, D))

    n_out = x_patient.shape[0] - 1
    grid = (pl.cdiv(n_out, TM),)
    wspec = pl.BlockSpec((D, D), lambda i: (0, 0))
    bspec = pl.BlockSpec((1, D), lambda i: (0, 0))
    out = pl.pallas_call(
        _fused_rows,
        grid=grid,
        in_specs=[
            pl.BlockSpec((TM, D), lambda i: (i, 0)),
            wspec, bspec, wspec, wspec, bspec,
        ],
        out_specs=pl.BlockSpec((TM, D), lambda i: (i, 0)),
        out_shape=jax.ShapeDtypeStruct((n_out, D), jnp.float32),
        compiler_params=pltpu.CompilerParams(
            dimension_semantics=("parallel",)),
    )(x_patient, W_in.astype(jnp.bfloat16), b_in.reshape(1, D),
      wro_bf16, w_comb, b_comb)
    return out


# speedup vs baseline: 1.0792x; 1.0792x over previous
import jax
import jax.numpy as jnp
from jax.experimental import pallas as pl
from jax.experimental.pallas import tpu as pltpu

D = 256
TM = 512


def _fused_rows(x_ref, wcat_ref, bcat_ref, wro_ref, bro_ref, o_ref):
    x = x_ref[...].astype(jnp.bfloat16)
    y = jax.lax.dot_general(x, wcat_ref[...], (((1,), (0,)), ((), ())),
                            preferred_element_type=jnp.float32)
    y += bcat_ref[...]
    s = jnp.tanh(y[:, :D]) + y[:, D:]
    o = jax.lax.dot_general(s.astype(jnp.bfloat16), wro_ref[...],
                            (((1,), (1,)), ((), ())),
                            preferred_element_type=jnp.float32)
    o_ref[...] = o + bro_ref[...]


def kernel(x_patient, x_se, edge_index, W_in, b_in, W_se, b_se, W_cl, b_cl,
           W_ro, b_ro, Wl_0_pse, bl_0_pse, Wr_0_pse, Wl_0_rev, bl_0_rev,
           Wr_0_rev, Wl_1_pse, bl_1_pse, Wr_1_pse, Wl_1_rev, bl_1_rev,
           Wr_1_rev):
    w_cat = jnp.concatenate([W_in.T, W_cl.T], axis=1).astype(jnp.bfloat16)
    b_cat = jnp.concatenate([b_in, b_cl]).reshape(1, 2 * D)
    n_out = x_patient.shape[0] - 1
    grid = (pl.cdiv(n_out, TM),)
    out = pl.pallas_call(
        _fused_rows,
        grid=grid,
        in_specs=[
            pl.BlockSpec((TM, D), lambda i: (i, 0)),
            pl.BlockSpec((D, 2 * D), lambda i: (0, 0)),
            pl.BlockSpec((1, 2 * D), lambda i: (0, 0)),
            pl.BlockSpec((D, D), lambda i: (0, 0)),
            pl.BlockSpec((1, D), lambda i: (0, 0)),
        ],
        out_specs=pl.BlockSpec((TM, D), lambda i: (i, 0)),
        out_shape=jax.ShapeDtypeStruct((n_out, D), jnp.float32),
        compiler_params=pltpu.CompilerParams(
            dimension_semantics=("parallel",)),
    )(x_patient, w_cat, b_cat, W_ro.astype(jnp.bfloat16), b_ro.reshape(1, D))
    return out


# speedup vs baseline: 1.3355x; 1.2375x over previous
import jax
import jax.numpy as jnp
from jax.experimental import pallas as pl
from jax.experimental.pallas import tpu as pltpu

D = 256
TM = 512

_DNT = (((1,), (1,)), ((), ()))


def _fused_rows(x_ref, win_ref, bin_ref, wcl_ref, bcl_ref, wro_ref, bro_ref,
                o_ref):
    x = x_ref[...].astype(jnp.bfloat16)
    t = jnp.tanh(jax.lax.dot_general(
        x, win_ref[...].astype(jnp.bfloat16), _DNT,
        preferred_element_type=jnp.float32) + bin_ref[...])
    h = jax.lax.dot_general(
        x, wcl_ref[...].astype(jnp.bfloat16), _DNT,
        preferred_element_type=jnp.float32) + bcl_ref[...]
    s = (t + h).astype(jnp.bfloat16)
    o = jax.lax.dot_general(
        s, wro_ref[...].astype(jnp.bfloat16), _DNT,
        preferred_element_type=jnp.float32)
    o_ref[...] = o + bro_ref[...]


def kernel(x_patient, x_se, edge_index, W_in, b_in, W_se, b_se, W_cl, b_cl,
           W_ro, b_ro, Wl_0_pse, bl_0_pse, Wr_0_pse, Wl_0_rev, bl_0_rev,
           Wr_0_rev, Wl_1_pse, bl_1_pse, Wr_1_pse, Wl_1_rev, bl_1_rev,
           Wr_1_rev):
    n_out = x_patient.shape[0] - 1
    grid = (pl.cdiv(n_out, TM),)
    wspec = pl.BlockSpec((D, D), lambda i: (0, 0))
    bspec = pl.BlockSpec((1, D), lambda i: (0, 0))
    out = pl.pallas_call(
        _fused_rows,
        grid=grid,
        in_specs=[
            pl.BlockSpec((TM, D), lambda i: (i, 0)),
            wspec, bspec, wspec, bspec, wspec, bspec,
        ],
        out_specs=pl.BlockSpec((TM, D), lambda i: (i, 0)),
        out_shape=jax.ShapeDtypeStruct((n_out, D), jnp.float32),
        compiler_params=pltpu.CompilerParams(
            dimension_semantics=("parallel",)),
    )(x_patient, W_in, b_in.reshape(1, D), W_cl, b_cl.reshape(1, D),
      W_ro, b_ro.reshape(1, D))
    return out


# speedup vs baseline: 2.3953x; 1.7936x over previous
import jax
import jax.numpy as jnp
from jax.experimental import pallas as pl
from jax.experimental.pallas import tpu as pltpu

D = 256
TM = 2048

_DNT = (((1,), (1,)), ((), ()))


def _fused_rows(x_ref, win_ref, bin_ref, wcl_ref, bcl_ref, wro_ref, bro_ref,
                o_ref):
    x = x_ref[...]
    t = jnp.tanh(jax.lax.dot_general(
        x, win_ref[...], _DNT,
        preferred_element_type=jnp.float32) + bin_ref[...])
    h = jax.lax.dot_general(
        x, wcl_ref[...], _DNT,
        preferred_element_type=jnp.float32) + bcl_ref[...]
    s = t + h
    o = jax.lax.dot_general(
        s, wro_ref[...], _DNT,
        preferred_element_type=jnp.float32)
    o_ref[...] = o + bro_ref[...]


def kernel(x_patient, x_se, edge_index, W_in, b_in, W_se, b_se, W_cl, b_cl,
           W_ro, b_ro, Wl_0_pse, bl_0_pse, Wr_0_pse, Wl_0_rev, bl_0_rev,
           Wr_0_rev, Wl_1_pse, bl_1_pse, Wr_1_pse, Wl_1_rev, bl_1_rev,
           Wr_1_rev):
    n_out = x_patient.shape[0] - 1
    grid = (pl.cdiv(n_out, TM),)
    wspec = pl.BlockSpec((D, D), lambda i: (0, 0))
    bspec = pl.BlockSpec((1, D), lambda i: (0, 0))
    out = pl.pallas_call(
        _fused_rows,
        grid=grid,
        in_specs=[
            pl.BlockSpec((TM, D), lambda i: (i, 0)),
            wspec, bspec, wspec, bspec, wspec, bspec,
        ],
        out_specs=pl.BlockSpec((TM, D), lambda i: (i, 0)),
        out_shape=jax.ShapeDtypeStruct((n_out, D), jnp.float32),
        compiler_params=pltpu.CompilerParams(
            dimension_semantics=("parallel",)),
    )(x_patient, W_in, b_in.reshape(1, D), W_cl, b_cl.reshape(1, D),
      W_ro, b_ro.reshape(1, D))
    return out


# speedup vs baseline: 2.5681x; 1.0722x over previous
import jax
import jax.numpy as jnp
from jax.experimental import pallas as pl
from jax.experimental.pallas import tpu as pltpu

D = 256
TM = 2560

_DNT = (((1,), (1,)), ((), ()))


def _fused_rows(x_ref, win_ref, bin_ref, wcl_ref, bcl_ref, wro_ref, bro_ref,
                o_ref):
    x = x_ref[...]
    t = jnp.tanh(jax.lax.dot_general(
        x, win_ref[...], _DNT,
        preferred_element_type=jnp.float32) + bin_ref[...])
    h = jax.lax.dot_general(
        x, wcl_ref[...], _DNT,
        preferred_element_type=jnp.float32) + bcl_ref[...]
    s = t + h
    o = jax.lax.dot_general(
        s, wro_ref[...], _DNT,
        preferred_element_type=jnp.float32)
    o_ref[...] = o + bro_ref[...]


def kernel(x_patient, x_se, edge_index, W_in, b_in, W_se, b_se, W_cl, b_cl,
           W_ro, b_ro, Wl_0_pse, bl_0_pse, Wr_0_pse, Wl_0_rev, bl_0_rev,
           Wr_0_rev, Wl_1_pse, bl_1_pse, Wr_1_pse, Wl_1_rev, bl_1_rev,
           Wr_1_rev):
    n_out = x_patient.shape[0] - 1
    grid = (pl.cdiv(n_out, TM),)
    wspec = pl.BlockSpec((D, D), lambda i: (0, 0))
    bspec = pl.BlockSpec((1, D), lambda i: (0, 0))
    out = pl.pallas_call(
        _fused_rows,
        grid=grid,
        in_specs=[
            pl.BlockSpec((TM, D), lambda i: (i, 0)),
            wspec, bspec, wspec, bspec, wspec, bspec,
        ],
        out_specs=pl.BlockSpec((TM, D), lambda i: (i, 0)),
        out_shape=jax.ShapeDtypeStruct((n_out, D), jnp.float32),
        compiler_params=pltpu.CompilerParams(
            dimension_semantics=("parallel",)),
    )(x_patient, W_in, b_in.reshape(1, D), W_cl, b_cl.reshape(1, D),
      W_ro, b_ro.reshape(1, D))
    return out


# speedup vs baseline: 2.6974x; 1.0503x over previous
import jax
import jax.numpy as jnp
from jax.experimental import pallas as pl
from jax.experimental.pallas import tpu as pltpu

D = 256
TM = 5120

_DNT = (((1,), (1,)), ((), ()))


def _fused_rows(x_ref, win_ref, bin_ref, wcl_ref, bcl_ref, wro_ref, bro_ref,
                o_ref):
    x = x_ref[...]
    t = jnp.tanh(jax.lax.dot_general(
        x, win_ref[...], _DNT,
        preferred_element_type=jnp.float32) + bin_ref[...])
    h = jax.lax.dot_general(
        x, wcl_ref[...], _DNT,
        preferred_element_type=jnp.float32) + bcl_ref[...]
    s = t + h
    o = jax.lax.dot_general(
        s, wro_ref[...], _DNT,
        preferred_element_type=jnp.float32)
    o_ref[...] = o + bro_ref[...]


def kernel(x_patient, x_se, edge_index, W_in, b_in, W_se, b_se, W_cl, b_cl,
           W_ro, b_ro, Wl_0_pse, bl_0_pse, Wr_0_pse, Wl_0_rev, bl_0_rev,
           Wr_0_rev, Wl_1_pse, bl_1_pse, Wr_1_pse, Wl_1_rev, bl_1_rev,
           Wr_1_rev):
    n_out = x_patient.shape[0] - 1
    grid = (pl.cdiv(n_out, TM),)
    wspec = pl.BlockSpec((D, D), lambda i: (0, 0))
    bspec = pl.BlockSpec((1, D), lambda i: (0, 0))
    out = pl.pallas_call(
        _fused_rows,
        grid=grid,
        in_specs=[
            pl.BlockSpec((TM, D), lambda i: (i, 0)),
            wspec, bspec, wspec, bspec, wspec, bspec,
        ],
        out_specs=pl.BlockSpec((TM, D), lambda i: (i, 0)),
        out_shape=jax.ShapeDtypeStruct((n_out, D), jnp.float32),
        compiler_params=pltpu.CompilerParams(
            dimension_semantics=("parallel",)),
    )(x_patient, W_in, b_in.reshape(1, D), W_cl, b_cl.reshape(1, D),
      W_ro, b_ro.reshape(1, D))
    return out
